# baseline (speedup 1.0000x reference)
import functools

import jax
import jax.numpy as jnp
from jax import lax
from jax.experimental import pallas as pl
from jax.experimental.pallas import tpu as pltpu
from jax.experimental.pallas import tpu_sc as plsc

_NC = 2
_NS = 16
_NW = _NC * _NS
_CHUNK = 128
_NB = 3
_NI = 6


def _segment_sum_partials(x, edge_index):
    n, d = x.shape
    e = edge_index.shape[1]
    assert e % _CHUNK == 0 and d % 16 == 0
    n_chunks = e // _CHUNK
    row_step = (n // _NS) // 8 * 8
    row_win = n - row_step * (_NS - 1)
    assert row_win % 8 == 0 and row_win >= row_step

    mesh = plsc.VectorSubcoreMesh(core_axis_name="c", subcore_axis_name="s")

    @functools.partial(
        pl.kernel,
        mesh=mesh,
        out_type=[jax.ShapeDtypeStruct((n, d), jnp.float32),
                  jax.ShapeDtypeStruct((n, d), jnp.float32)],
        scratch_types=[
            pltpu.VMEM((_NI, 2, _CHUNK), jnp.int32),
            pltpu.VMEM((_NB, _CHUNK, d), jnp.float32),
            pltpu.VMEM_SHARED((n, d), jnp.float32),
            pltpu.SemaphoreType.DMA((_NI,)),
            pltpu.SemaphoreType.DMA((_NB,)),
            pltpu.SemaphoreType.DMA((_NB,)),
        ],
    )
    def seg_kernel(x_hbm, ei_hbm, out0_hbm, out1_hbm, idx_v, rows_v, agg_sh,
                   isem, gsem, ssem):
        cid = lax.axis_index("c")
        sid = lax.axis_index("s")
        wid = sid * _NC + cid

        zvec = jnp.zeros((16,), jnp.float32)

        def zero_row(r, carry):
            for c in range(d // 16):
                rows_v[0, r, pl.ds(c * 16, 16)] = zvec
            return carry

        c0 = (n_chunks * wid) // _NW
        n_my = (n_chunks * (wid + 1)) // _NW - c0

        def load_idx(t, islot):
            return pltpu.make_async_copy(
                ei_hbm.at[:, pl.ds((c0 + t) * _CHUNK, _CHUNK)],
                idx_v.at[islot], isem.at[islot])

        def gather(bslot, islot):
            return pltpu.make_async_copy(x_hbm.at[idx_v.at[islot, 0]],
                                         rows_v.at[bslot], gsem.at[bslot])

        def scatter(bslot, islot):
            return pltpu.make_async_copy(rows_v.at[bslot],
                                         agg_sh.at[idx_v.at[islot, 1]],
                                         ssem.at[bslot])

        for j in range(_NI // 2):
            load_idx(j, j).start()

        lax.fori_loop(0, _CHUNK, zero_row, 0)
        row0 = sid * row_step
        assert row_win % _CHUNK == 0
        nzcopy = row_win // _CHUNK
        zsems = [isem.at[_NI // 2 + k] for k in range(_NI - _NI // 2)]
        zsems += [ssem.at[k] for k in range(nzcopy - len(zsems))]

        def zcopy(k):
            return pltpu.make_async_copy(
                rows_v.at[0], agg_sh.at[pl.ds(row0 + k * _CHUNK, _CHUNK)],
                zsems[k])

        for k in range(nzcopy):
            zcopy(k).start()
        load_idx(0, 0).wait()
        for k in range(nzcopy):
            zcopy(k).wait()
        gather(0, 0).start()
        plsc.subcore_barrier()

        def group(g, carry):
            for u in range(_NI):
                t = g * _NI + u
                b = u % _NB
                ib = u % _NI

                t1 = t + 1
                b1 = (u + 1) % _NB
                ib1 = (u + 1) % _NI

                @pl.when(t < n_my)
                def _():
                    gather(b, ib).wait()

                @pl.when(jnp.logical_and(t1 < n_my, t1 >= _NB))
                def _():
                    scatter(b1, ib1).wait()

                @pl.when(t1 < n_my)
                def _():
                    load_idx(t1, ib1).wait()
                    gather(b1, ib1).start()

                @pl.when(t < n_my)
                def _():
                    scatter(b, ib).start(add=True)
                t3 = t + _NI // 2
                ib3 = (u + _NI // 2) % _NI

                @pl.when(t3 < n_my)
                def _():
                    load_idx(t3, ib3).start()
            return carry

        lax.fori_loop(0, (n_my + _NI - 1) // _NI, group, 0)
        for b in range(_NB):
            scatter(b, b).wait()
        plsc.subcore_barrier()

        @pl.when(cid == 0)
        def _():
            pltpu.sync_copy(agg_sh.at[pl.ds(row0, row_win)],
                            out0_hbm.at[pl.ds(row0, row_win)])

        @pl.when(cid == 1)
        def _():
            pltpu.sync_copy(agg_sh.at[pl.ds(row0, row_win)],
                            out1_hbm.at[pl.ds(row0, row_win)])

    return seg_kernel(x, edge_index)


_BLK = 2000


def _mlp_layer(x, p0, p1, w1, b1, w2, b2, g, bt):
    n, d = x.shape
    assert n % _BLK == 0
    inv_std = float(1.0 / (1.0 + 1e-5) ** 0.5)

    def body(x_ref, p0_ref, p1_ref, w1_ref, b1_ref, w2_ref, b2_ref, g_ref,
             bt_ref, o_ref):
        h = x_ref[...] + p0_ref[...] + p1_ref[...]
        h = lax.dot(h, w1_ref[...],
                    preferred_element_type=jnp.float32) + b1_ref[...]
        h = jnp.maximum(h, 0.0)
        h = lax.dot(h, w2_ref[...],
                    preferred_element_type=jnp.float32) + b2_ref[...]
        h = jnp.maximum(h, 0.0)
        o_ref[...] = g_ref[...] * (h * inv_std) + bt_ref[...]

    blk = pl.BlockSpec((_BLK, d), lambda i: (i, 0))
    wblk = pl.BlockSpec((d, d), lambda i: (0, 0))
    vblk = pl.BlockSpec((1, d), lambda i: (0, 0))
    return pl.pallas_call(
        body,
        grid=(n // _BLK,),
        in_specs=[blk, blk, blk, wblk, vblk, wblk, vblk, vblk, vblk],
        out_specs=blk,
        out_shape=jax.ShapeDtypeStruct((n, d), jnp.float32),
    )(x, p0, p1, w1, b1.reshape(1, d), w2, b2.reshape(1, d),
      g.reshape(1, d), bt.reshape(1, d))


def kernel(x, edge_index, W1, b1, W2, b2, gamma, beta):
    num_layers = W1.shape[0]
    out = x
    recs = []
    for i in range(num_layers):
        p0, p1 = _segment_sum_partials(out, edge_index)
        out = _mlp_layer(out, p0, p1, W1[i], b1[i], W2[i], b2[i],
                         gamma[i], beta[i])
        recs.append(out)
    return jnp.concatenate(recs, axis=-1)


# speedup vs baseline: 1.0173x; 1.0173x over previous
import functools

import jax
import jax.numpy as jnp
from jax import lax
from jax.experimental import pallas as pl
from jax.experimental.pallas import tpu as pltpu
from jax.experimental.pallas import tpu_sc as plsc

_NC = 2
_NS = 16
_NW = _NC * _NS
_CHUNK = 128
_NB = 3
_NI = 6


def _segment_sum_partials(x, edge_index):
    n, d = x.shape
    e = edge_index.shape[1]
    assert e % _CHUNK == 0 and d % 16 == 0
    n_chunks = e // _CHUNK
    row_step = (n // _NS) // 8 * 8
    row_win = n - row_step * (_NS - 1)
    assert row_win % 8 == 0 and row_win >= row_step

    mesh = plsc.VectorSubcoreMesh(core_axis_name="c", subcore_axis_name="s")

    @functools.partial(
        pl.kernel,
        mesh=mesh,
        out_type=[jax.ShapeDtypeStruct((n, d), jnp.float32),
                  jax.ShapeDtypeStruct((n, d), jnp.float32)],
        scratch_types=[
            pltpu.VMEM((_NI, 2, _CHUNK), jnp.int32),
            pltpu.VMEM((_NB, _CHUNK, d), jnp.float32),
            pltpu.VMEM_SHARED((n, d), jnp.float32),
            pltpu.SemaphoreType.DMA((_NI,)),
            pltpu.SemaphoreType.DMA((_NB,)),
            pltpu.SemaphoreType.DMA((_NB,)),
        ],
    )
    def seg_kernel(x_hbm, ei_hbm, out0_hbm, out1_hbm, idx_v, rows_v, agg_sh,
                   isem, gsem, ssem):
        cid = lax.axis_index("c")
        sid = lax.axis_index("s")
        wid = sid * _NC + cid

        zvec = jnp.zeros((16,), jnp.float32)

        def zero_row(r, carry):
            for c in range(d // 16):
                rows_v[0, r, pl.ds(c * 16, 16)] = zvec
            return carry

        c0 = (n_chunks * wid) // _NW
        n_my = (n_chunks * (wid + 1)) // _NW - c0

        def load_idx(t, islot):
            return pltpu.make_async_copy(
                ei_hbm.at[:, pl.ds((c0 + t) * _CHUNK, _CHUNK)],
                idx_v.at[islot], isem.at[islot])

        def gather(bslot, islot):
            return pltpu.make_async_copy(x_hbm.at[idx_v.at[islot, 0]],
                                         rows_v.at[bslot], gsem.at[bslot])

        def scatter(bslot, islot):
            return pltpu.make_async_copy(rows_v.at[bslot],
                                         agg_sh.at[idx_v.at[islot, 1]],
                                         ssem.at[bslot])

        for j in range(_NI // 2):
            load_idx(j, j).start()

        lax.fori_loop(0, _CHUNK, zero_row, 0)
        row0 = sid * row_step
        assert row_win % _CHUNK == 0
        nzcopy = row_win // _CHUNK
        zsems = [isem.at[_NI // 2 + k] for k in range(_NI - _NI // 2)]
        zsems += [ssem.at[k] for k in range(nzcopy - len(zsems))]

        def zcopy(k):
            return pltpu.make_async_copy(
                rows_v.at[0], agg_sh.at[pl.ds(row0 + k * _CHUNK, _CHUNK)],
                zsems[k])

        for k in range(nzcopy):
            zcopy(k).start()
        load_idx(0, 0).wait()
        for k in range(nzcopy):
            zcopy(k).wait()
        gather(0, 0).start()
        plsc.subcore_barrier()

        def group(g, carry):
            for u in range(_NI):
                t = g * _NI + u
                b = u % _NB
                ib = u % _NI

                t1 = t + 1
                b1 = (u + 1) % _NB
                ib1 = (u + 1) % _NI

                @pl.when(t < n_my)
                def _():
                    gather(b, ib).wait()

                @pl.when(jnp.logical_and(t1 < n_my, t1 >= _NB))
                def _():
                    scatter(b1, ib1).wait()

                @pl.when(t1 < n_my)
                def _():
                    load_idx(t1, ib1).wait()
                    gather(b1, ib1).start()

                @pl.when(t < n_my)
                def _():
                    scatter(b, ib).start(add=True)
                t3 = t + _NI // 2
                ib3 = (u + _NI // 2) % _NI

                @pl.when(t3 < n_my)
                def _():
                    load_idx(t3, ib3).start()
            return carry

        lax.fori_loop(0, (n_my + _NI - 1) // _NI, group, 0)
        for b in range(_NB):
            scatter(b, b).wait()
        plsc.subcore_barrier()

        @pl.when(cid == 0)
        def _():
            pltpu.sync_copy(agg_sh.at[pl.ds(row0, row_win)],
                            out0_hbm.at[pl.ds(row0, row_win)])

        @pl.when(cid == 1)
        def _():
            pltpu.sync_copy(agg_sh.at[pl.ds(row0, row_win)],
                            out1_hbm.at[pl.ds(row0, row_win)])

    return seg_kernel(x, edge_index)


_BLK = 2000


def _mlp_layer(x, p0, p1, w1, b1, w2, b2, g, bt, prev=None):
    n, d = x.shape
    assert n % _BLK == 0
    inv_std = float(1.0 / (1.0 + 1e-5) ** 0.5)
    prev = list(prev) if prev else []
    n_out = len(prev) + 1

    def body(*refs):
        (x_ref, p0_ref, p1_ref, w1_ref, b1_ref, w2_ref, b2_ref, g_ref,
         bt_ref), prev_refs, o_ref = refs[:9], refs[9:-1], refs[-1]
        h = x_ref[...] + p0_ref[...] + p1_ref[...]
        h = lax.dot(h, w1_ref[...],
                    preferred_element_type=jnp.float32) + b1_ref[...]
        h = jnp.maximum(h, 0.0)
        h = lax.dot(h, w2_ref[...],
                    preferred_element_type=jnp.float32) + b2_ref[...]
        h = jnp.maximum(h, 0.0)
        h = g_ref[...] * (h * inv_std) + bt_ref[...]
        for k, p_ref in enumerate(prev_refs):
            o_ref[:, k * d:(k + 1) * d] = p_ref[...]
        o_ref[:, len(prev_refs) * d:] = h

    blk = pl.BlockSpec((_BLK, d), lambda i: (i, 0))
    wblk = pl.BlockSpec((d, d), lambda i: (0, 0))
    vblk = pl.BlockSpec((1, d), lambda i: (0, 0))
    return pl.pallas_call(
        body,
        grid=(n // _BLK,),
        in_specs=[blk, blk, blk, wblk, vblk, wblk, vblk, vblk, vblk]
                 + [blk] * len(prev),
        out_specs=pl.BlockSpec((_BLK, n_out * d), lambda i: (i, 0)),
        out_shape=jax.ShapeDtypeStruct((n, n_out * d), jnp.float32),
    )(x, p0, p1, w1, b1.reshape(1, d), w2, b2.reshape(1, d),
      g.reshape(1, d), bt.reshape(1, d), *prev)


def kernel(x, edge_index, W1, b1, W2, b2, gamma, beta):
    num_layers = W1.shape[0]
    out = x
    recs = []
    for i in range(num_layers):
        p0, p1 = _segment_sum_partials(out, edge_index)
        last = i == num_layers - 1
        out = _mlp_layer(out, p0, p1, W1[i], b1[i], W2[i], b2[i],
                         gamma[i], beta[i], prev=recs if last else None)
        if not last:
            recs.append(out)
    return out
